# baseline (device time: 67962 ns/iter reference)
import jax
import jax.numpy as jnp
from jax import lax
from jax.experimental import pallas as pl
from jax.experimental.pallas import tpu as pltpu

N_DEV = 4
EPS = 1e-5
RB = 512


def _stats_and_copy(x):
    M, Nl = x.shape
    RB = 1024
    nblk = M // RB

    def body(x_ref, xb_ref, s_ref, q_ref):
        g = pl.program_id(0)
        xcb = x_ref[:, :].astype(jnp.bfloat16)
        xb_ref[:, :] = xcb
        ones_row = jnp.ones((1, Nl), dtype=jnp.bfloat16)
        s_ref[pl.ds(g, 1), :] = lax.dot_general(
            ones_row, xcb, (((1,), (1,)), ((), ())),
            preferred_element_type=jnp.float32,
        )
        q_ref[pl.ds(g, 1), :] = lax.dot_general(
            ones_row, xcb * xcb, (((1,), (1,)), ((), ())),
            preferred_element_type=jnp.float32,
        )

    return pl.pallas_call(
        body,
        grid=(nblk,),
        in_specs=[pl.BlockSpec((RB, Nl), lambda g: (g, 0))],
        out_specs=[
            pl.BlockSpec((RB, Nl), lambda g: (g, 0)),
            pl.BlockSpec((nblk, RB), lambda g: (0, 0)),
            pl.BlockSpec((nblk, RB), lambda g: (0, 0)),
        ],
        out_shape=[
            jax.ShapeDtypeStruct((M, Nl), jnp.bfloat16),
            jax.ShapeDtypeStruct((nblk, RB), jnp.float32),
            jax.ShapeDtypeStruct((nblk, RB), jnp.float32),
        ],
    )(x)


def _allreduce_stats(s_t, q_t, n_global, M):
    nblk, RB = s_t.shape

    def body(s_ref, q_ref, out_ref, comm, send_sems, recv_sems):
        my = lax.axis_index("i")

        barrier_sem = pltpu.get_barrier_semaphore()
        for off in (1, 2, 3):
            pl.semaphore_signal(
                barrier_sem, inc=1,
                device_id=(lax.rem(my + off, N_DEV),),
                device_id_type=pl.DeviceIdType.MESH,
            )
        pl.semaphore_wait(barrier_sem, 3)

        comm[my, 0:nblk, :] = s_ref[:, :]
        comm[my, nblk:, :] = q_ref[:, :]
        sends = []
        for off in (1, 2, 3):
            r = pltpu.make_async_remote_copy(
                src_ref=comm.at[my],
                dst_ref=comm.at[my],
                send_sem=send_sems.at[off],
                recv_sem=recv_sems.at[my],
                device_id=(lax.rem(my + off, N_DEV),),
                device_id_type=pl.DeviceIdType.MESH,
            )
            r.start()
            sends.append(r)

        acc = comm[my]
        for off in (1, 2, 3):
            src = lax.rem(my - off + N_DEV, N_DEV)
            recv = pltpu.make_async_remote_copy(
                src_ref=comm.at[src],
                dst_ref=comm.at[src],
                send_sem=send_sems.at[0],
                recv_sem=recv_sems.at[src],
                device_id=(my,),
                device_id_type=pl.DeviceIdType.MESH,
            )
            recv.wait_recv()
            acc = acc + comm[src]

        mean_rows = acc[0:nblk, :] / n_global
        var_rows = acc[nblk:, :] / n_global - mean_rows * mean_rows
        rstd_rows = lax.rsqrt(var_rows + EPS)
        mr = jnp.concatenate([mean_rows, rstd_rows], axis=0)
        eye = jnp.eye(RB, dtype=jnp.float32)
        cols = lax.dot_general(
            eye, mr, (((1,), (1,)), ((), ())),
            preferred_element_type=jnp.float32,
        )
        colsb = cols.astype(jnp.bfloat16)
        for b in range(nblk):
            out_ref[pl.ds(b * RB, RB), :] = jnp.concatenate(
                [colsb[:, b:b + 1], colsb[:, nblk + b:nblk + b + 1]], axis=1
            )

        for r in sends:
            r.wait_send()

    return pl.pallas_call(
        body,
        out_shape=jax.ShapeDtypeStruct((M, 2), jnp.bfloat16),
        in_specs=[
            pl.BlockSpec(memory_space=pltpu.VMEM),
            pl.BlockSpec(memory_space=pltpu.VMEM),
        ],
        out_specs=pl.BlockSpec(memory_space=pltpu.VMEM),
        scratch_shapes=[
            pltpu.VMEM((N_DEV, 2 * nblk, RB), jnp.float32),
            pltpu.SemaphoreType.DMA((N_DEV,)),
            pltpu.SemaphoreType.DMA((N_DEV,)),
        ],
        compiler_params=pltpu.CompilerParams(collective_id=0),
    )(s_t, q_t)


def _normalize(xb16, mr2, gamma2, beta2):
    M, Nl = xb16.shape
    RB = 1024
    nblk = M // RB

    def body(x_ref, mr_ref, g_ref, b_ref, o_ref):
        m_col = mr_ref[:, 0:1]
        r_col = mr_ref[:, 1:2]
        xb = x_ref[:, :]
        gb = g_ref[:, :].astype(jnp.bfloat16)
        bb = b_ref[:, :].astype(jnp.bfloat16)
        o_ref[:, :] = (xb - m_col) * r_col * gb + bb

    return pl.pallas_call(
        body,
        grid=(nblk,),
        in_specs=[
            pl.BlockSpec((RB, Nl), lambda g: (g, 0)),
            pl.BlockSpec((RB, 2), lambda g: (g, 0)),
            pl.BlockSpec((1, Nl), lambda g: (0, 0)),
            pl.BlockSpec((1, Nl), lambda g: (0, 0)),
        ],
        out_specs=pl.BlockSpec((RB, Nl), lambda g: (g, 0)),
        out_shape=jax.ShapeDtypeStruct((M, Nl), jnp.bfloat16),
    )(xb16, mr2, gamma2, beta2)


def kernel(x, gamma, beta):
    M, Nl = x.shape
    n_global = Nl * N_DEV

    xb16, s_t, q_t = _stats_and_copy(x)
    mr2 = _allreduce_stats(s_t, q_t, n_global, M)
    return _normalize(xb16, mr2, gamma.reshape(1, Nl), beta.reshape(1, Nl))
